# baseline (speedup 1.0000x reference)
import functools
import math

import jax
import jax.numpy as jnp
from jax import lax
from jax.experimental import pallas as pl
from jax.experimental.pallas import tpu as pltpu
from jax.experimental.pallas import tpu_sc as plsc

_NUM_CORES = 2
_NUM_SUBCORES = 16
_NUM_WORKERS = _NUM_CORES * _NUM_SUBCORES
_LANES = 16

_CHUNK = 32


def _build_sc_kernel(vocab, d, n_rows, seq_len):
    segs = d // _LANES
    rows_per_w = n_rows // _NUM_WORKERS
    n_chunks = rows_per_w // _CHUNK
    scale = float(math.sqrt(float(d)))
    neg_ln_base = -math.log(10000.0)

    mesh = plsc.VectorSubcoreMesh(core_axis_name="c", subcore_axis_name="s")

    @functools.partial(
        pl.kernel,
        mesh=mesh,
        out_type=jax.ShapeDtypeStruct((n_rows, d), jnp.float32),
        scratch_types=[
            pltpu.VMEM((n_chunks, _CHUNK), jnp.int32),
            pltpu.VMEM((d,), jnp.float32),
            pltpu.VMEM((2, _CHUNK, d), jnp.float32),
            pltpu.SemaphoreType.DMA,
            pltpu.SemaphoreType.DMA,
            pltpu.SemaphoreType.DMA,
            pltpu.SemaphoreType.DMA,
        ],
    )
    def sc_kernel(table_hbm, idx_hbm, out_hbm, idx_v, invf_v, rows_v,
                  gsem0, gsem1, osem0, osem1):
        wid = lax.axis_index("s") * _NUM_CORES + lax.axis_index("c")
        base = wid * rows_per_w

        pltpu.sync_copy(idx_hbm.at[wid], idx_v)

        @pl.loop(0, segs)
        def _(s):
            ji = lax.iota(jnp.int32, _LANES) + s * _LANES
            jf = ji.astype(jnp.float32)
            mf = (ji & 1).astype(jnp.float32)
            e = (jf - mf) * (1.0 / float(d))
            invf_v[pl.ds(s * _LANES, _LANES)] = jnp.exp(e * neg_ln_base)

        gsems = (gsem0, gsem1)
        osems = (osem0, osem1)

        def mk_gather(c, buf):
            return pltpu.make_async_copy(
                table_hbm.at[idx_v.at[c]], rows_v.at[buf], gsems[buf])

        def mk_out(c, buf):
            return pltpu.make_async_copy(
                rows_v.at[buf],
                out_hbm.at[pl.ds(base + c * _CHUNK, _CHUNK)],
                osems[buf])

        def compute(c, buf):
            posbase = (base + c * _CHUNK) % seq_len
            buf_ref = rows_v.at[buf]

            @pl.loop(0, segs)
            def _(s):
                col = s * _LANES
                invf = invf_v[pl.ds(col, _LANES)]

                @pl.loop(0, _CHUNK)
                def _(r):
                    posf = (posbase + r).astype(jnp.float32)
                    g = buf_ref[r, pl.ds(col, _LANES)]
                    buf_ref[r, pl.ds(col, _LANES)] = g * scale + posf * invf

        out_copies = [None] * n_chunks
        gather_copies = [None] * n_chunks
        gather_copies[0] = mk_gather(0, 0)
        gather_copies[0].start()
        for c in range(n_chunks):
            buf = c % 2
            if c + 1 < n_chunks:
                if c >= 1:
                    out_copies[c - 1].wait()
                gather_copies[c + 1] = mk_gather(c + 1, 1 - buf)
                gather_copies[c + 1].start()
            gather_copies[c].wait()
            compute(c, buf)
            out_copies[c] = mk_out(c, buf)
            out_copies[c].start()
        out_copies[n_chunks - 2].wait()
        out_copies[n_chunks - 1].wait()

    return sc_kernel


def kernel(sequence, embedding_table):
    b, s = sequence.shape
    vocab, d = embedding_table.shape
    n_rows = b * s
    idx = sequence.reshape(_NUM_WORKERS, n_rows // (_NUM_WORKERS * _CHUNK),
                           _CHUNK)
    sc = _build_sc_kernel(vocab, d, n_rows, s)
    out = sc(embedding_table, idx)
    return out.reshape(b, s, d)


# speedup vs baseline: 2.9564x; 2.9564x over previous
import functools
import math

import jax
import jax.numpy as jnp
from jax import lax
from jax.experimental import pallas as pl
from jax.experimental.pallas import tpu as pltpu
from jax.experimental.pallas import tpu_sc as plsc

_NUM_CORES = 2
_NUM_SUBCORES = 16
_NUM_WORKERS = _NUM_CORES * _NUM_SUBCORES
_LANES = 16

_CHUNK = 32


def _build_sc_kernel(vocab, d, n_rows, seq_len):
    segs = d // _LANES
    rows_per_w = n_rows // _NUM_WORKERS
    n_chunks = rows_per_w // _CHUNK
    scale = float(math.sqrt(float(d)))
    neg_ln_base = -math.log(10000.0)

    mesh = plsc.VectorSubcoreMesh(core_axis_name="c", subcore_axis_name="s")

    @functools.partial(
        pl.kernel,
        mesh=mesh,
        out_type=jax.ShapeDtypeStruct((n_rows, d), jnp.float32),
        scratch_types=[
            pltpu.VMEM((n_chunks, _CHUNK), jnp.int32),
            pltpu.VMEM((d,), jnp.float32),
            pltpu.VMEM((2, _CHUNK, d), jnp.float32),
            pltpu.SemaphoreType.DMA,
            pltpu.SemaphoreType.DMA,
            pltpu.SemaphoreType.DMA,
            pltpu.SemaphoreType.DMA,
        ],
    )
    def sc_kernel(table_hbm, idx_hbm, out_hbm, idx_v, invf_v, rows_v,
                  gsem0, gsem1, osem0, osem1):
        wid = lax.axis_index("s") * _NUM_CORES + lax.axis_index("c")
        base = wid * rows_per_w

        pltpu.sync_copy(idx_hbm.at[wid], idx_v)

        @pl.loop(0, segs)
        def _(s):
            ji = lax.iota(jnp.int32, _LANES) + s * _LANES
            jf = ji.astype(jnp.float32)
            mf = (ji & 1).astype(jnp.float32)
            e = (jf - mf) * (1.0 / float(d))
            invf_v[pl.ds(s * _LANES, _LANES)] = jnp.exp(e * neg_ln_base)

        gsems = (gsem0, gsem1)
        osems = (osem0, osem1)

        def mk_gather(c, buf):
            return pltpu.make_async_copy(
                table_hbm.at[idx_v.at[c]], rows_v.at[buf], gsems[buf])

        def mk_out(c, buf):
            return pltpu.make_async_copy(
                rows_v.at[buf],
                out_hbm.at[pl.ds(base + c * _CHUNK, _CHUNK)],
                osems[buf])

        def compute(c, buf):
            posbase = (base + c * _CHUNK) % seq_len
            buf_ref = rows_v.at[buf]

            @pl.loop(0, segs)
            def _(s):
                col = s * _LANES
                invf = invf_v[pl.ds(col, _LANES)]

                @plsc.parallel_loop(0, _CHUNK, unroll=8)
                def _(r):
                    posf = (posbase + r).astype(jnp.float32)
                    g = buf_ref[r, pl.ds(col, _LANES)]
                    buf_ref[r, pl.ds(col, _LANES)] = g * scale + posf * invf

        out_copies = [None] * n_chunks
        gather_copies = [None] * n_chunks
        gather_copies[0] = mk_gather(0, 0)
        gather_copies[0].start()
        for c in range(n_chunks):
            buf = c % 2
            if c + 1 < n_chunks:
                if c >= 1:
                    out_copies[c - 1].wait()
                gather_copies[c + 1] = mk_gather(c + 1, 1 - buf)
                gather_copies[c + 1].start()
            gather_copies[c].wait()
            compute(c, buf)
            out_copies[c] = mk_out(c, buf)
            out_copies[c].start()
        out_copies[n_chunks - 2].wait()
        out_copies[n_chunks - 1].wait()

    return sc_kernel


def kernel(sequence, embedding_table):
    b, s = sequence.shape
    vocab, d = embedding_table.shape
    n_rows = b * s
    idx = sequence.reshape(_NUM_WORKERS, n_rows // (_NUM_WORKERS * _CHUNK),
                           _CHUNK)
    sc = _build_sc_kernel(vocab, d, n_rows, s)
    out = sc(embedding_table, idx)
    return out.reshape(b, s, d)


# speedup vs baseline: 3.0707x; 1.0387x over previous
import functools
import math

import jax
import jax.numpy as jnp
from jax import lax
from jax.experimental import pallas as pl
from jax.experimental.pallas import tpu as pltpu
from jax.experimental.pallas import tpu_sc as plsc

_NUM_CORES = 2
_NUM_SUBCORES = 16
_NUM_WORKERS = _NUM_CORES * _NUM_SUBCORES
_LANES = 16

_CHUNK = 32


def _build_sc_kernel(vocab, d, n_rows, seq_len):
    segs = d // _LANES
    rows_per_w = n_rows // _NUM_WORKERS
    n_chunks = rows_per_w // _CHUNK
    scale = float(math.sqrt(float(d)))
    neg_ln_base = -math.log(10000.0)

    mesh = plsc.VectorSubcoreMesh(core_axis_name="c", subcore_axis_name="s")

    @functools.partial(
        pl.kernel,
        mesh=mesh,
        out_type=jax.ShapeDtypeStruct((n_rows, d), jnp.float32),
        scratch_types=[
            pltpu.VMEM((n_chunks, _CHUNK), jnp.int32),
            pltpu.VMEM((d,), jnp.float32),
            pltpu.VMEM((3, _CHUNK, d), jnp.float32),
            pltpu.SemaphoreType.DMA,
            pltpu.SemaphoreType.DMA,
            pltpu.SemaphoreType.DMA,
            pltpu.SemaphoreType.DMA,
            pltpu.SemaphoreType.DMA,
            pltpu.SemaphoreType.DMA,
        ],
    )
    def sc_kernel(table_hbm, idx_hbm, out_hbm, idx_v, invf_v, rows_v,
                  gsem0, gsem1, gsem2, osem0, osem1, osem2):
        wid = lax.axis_index("s") * _NUM_CORES + lax.axis_index("c")
        base = wid * rows_per_w

        pltpu.sync_copy(idx_hbm.at[wid], idx_v)

        @pl.loop(0, segs)
        def _(s):
            ji = lax.iota(jnp.int32, _LANES) + s * _LANES
            jf = ji.astype(jnp.float32)
            mf = (ji & 1).astype(jnp.float32)
            e = (jf - mf) * (1.0 / float(d))
            invf_v[pl.ds(s * _LANES, _LANES)] = jnp.exp(e * neg_ln_base)

        gsems = (gsem0, gsem1, gsem2)
        osems = (osem0, osem1, osem2)

        def mk_gather(c, buf):
            return pltpu.make_async_copy(
                table_hbm.at[idx_v.at[c]], rows_v.at[buf], gsems[buf])

        def mk_out(c, buf):
            return pltpu.make_async_copy(
                rows_v.at[buf],
                out_hbm.at[pl.ds(base + c * _CHUNK, _CHUNK)],
                osems[buf])

        def compute(c, buf):
            posbase = (base + c * _CHUNK) % seq_len
            buf_ref = rows_v.at[buf]

            @pl.loop(0, segs)
            def _(s):
                col = s * _LANES
                invf = invf_v[pl.ds(col, _LANES)]

                @plsc.parallel_loop(0, _CHUNK, unroll=8)
                def _(r):
                    posf = (posbase + r).astype(jnp.float32)
                    g = buf_ref[r, pl.ds(col, _LANES)]
                    buf_ref[r, pl.ds(col, _LANES)] = g * scale + posf * invf

        nbuf = 3
        out_copies = [None] * n_chunks
        gather_copies = [None] * n_chunks
        gather_copies[0] = mk_gather(0, 0)
        gather_copies[0].start()
        for c in range(n_chunks):
            buf = c % nbuf
            if c + 1 < n_chunks:
                if c - 2 >= 0:
                    out_copies[c - 2].wait()
                gather_copies[c + 1] = mk_gather(c + 1, (c + 1) % nbuf)
                gather_copies[c + 1].start()
            gather_copies[c].wait()
            compute(c, buf)
            out_copies[c] = mk_out(c, buf)
            out_copies[c].start()
        for c in range(max(0, n_chunks - nbuf), n_chunks):
            out_copies[c].wait()

    return sc_kernel


def kernel(sequence, embedding_table):
    b, s = sequence.shape
    vocab, d = embedding_table.shape
    n_rows = b * s
    idx = sequence.reshape(_NUM_WORKERS, n_rows // (_NUM_WORKERS * _CHUNK),
                           _CHUNK)
    sc = _build_sc_kernel(vocab, d, n_rows, s)
    out = sc(embedding_table, idx)
    return out.reshape(b, s, d)
